# Initial kernel scaffold; baseline (speedup 1.0000x reference)
import jax
import jax.numpy as jnp
from jax.experimental import pallas as pl


def kernel(x, edge_attr, edge_index, WQ, bQ, WK, WE, bE, WV, Aw, VeRow):
    raise NotImplementedError("write your pallas kernel here")


# speedup vs baseline: 45.7705x; 45.7705x over previous
import functools

import jax
import jax.numpy as jnp
from jax import lax
from jax.experimental import pallas as pl
from jax.experimental.pallas import tpu as pltpu
from jax.experimental.pallas import tpu_sc as plsc

N_NODES = 10000
N_EDGES = 320000
IN_DIM = 128
H = 8
D = 16
HD = H * D
CLAMP = 5.0

f32 = jnp.float32


_ROWS_BLK = 1000


def _proj_body(x_ref, w_ref, b_ref, q_ref, k_ref, v_ref):
    y = jnp.dot(x_ref[...], w_ref[...], preferred_element_type=f32) + b_ref[...]
    q_ref[...] = y[:, :HD]
    k_ref[...] = y[:, HD:2 * HD]
    v_ref[...] = y[:, 2 * HD:]


def _proj(x, Wcat, bcat):
    n_blk = N_NODES // _ROWS_BLK
    return pl.pallas_call(
        _proj_body,
        grid=(n_blk,),
        in_specs=[
            pl.BlockSpec((_ROWS_BLK, IN_DIM), lambda i: (i, 0)),
            pl.BlockSpec((IN_DIM, 3 * HD), lambda i: (0, 0)),
            pl.BlockSpec((1, 3 * HD), lambda i: (0, 0)),
        ],
        out_specs=[
            pl.BlockSpec((_ROWS_BLK, HD), lambda i: (i, 0)),
            pl.BlockSpec((_ROWS_BLK, HD), lambda i: (i, 0)),
            pl.BlockSpec((_ROWS_BLK, HD), lambda i: (i, 0)),
        ],
        out_shape=[jax.ShapeDtypeStruct((N_NODES, HD), f32)] * 3,
    )(x, Wcat, bcat)


_NC = 2
_NS = 16
_NW = _NC * _NS
_EPW = N_EDGES // _NW
_GC = 80
_GN = _EPW // _GC

_sc_mesh = plsc.VectorSubcoreMesh(core_axis_name="c", subcore_axis_name="s")


@functools.partial(
    pl.kernel,
    out_type=[jax.ShapeDtypeStruct((N_EDGES, HD), f32)] * 3,
    mesh=_sc_mesh,
    scratch_types=[
        pltpu.VMEM((_GC,), jnp.int32),
        pltpu.VMEM((_GC,), jnp.int32),
        pltpu.VMEM((_GC, HD), f32),
        pltpu.VMEM((_GC, HD), f32),
        pltpu.VMEM((_GC, HD), f32),
        pltpu.SemaphoreType.DMA,
        pltpu.SemaphoreType.DMA,
        pltpu.SemaphoreType.DMA,
    ],
)
def _sc_gather(src_hbm, dst_hbm, q_hbm, k_hbm, v_hbm,
               ks_out, qd_out, vs_out,
               sidx, didx, kbuf, qbuf, vbuf, sem_k, sem_q, sem_v):
    w = lax.axis_index("s") * _NC + lax.axis_index("c")
    base0 = w * _EPW

    def body(i, _):
        base = base0 + i * _GC
        pltpu.sync_copy(src_hbm.at[pl.ds(base, _GC)], sidx)
        pltpu.sync_copy(dst_hbm.at[pl.ds(base, _GC)], didx)
        ck = pltpu.async_copy(k_hbm.at[sidx], kbuf, sem_k)
        cq = pltpu.async_copy(q_hbm.at[didx], qbuf, sem_q)
        cv = pltpu.async_copy(v_hbm.at[sidx], vbuf, sem_v)
        ck.wait()
        cq.wait()
        cv.wait()
        pltpu.sync_copy(kbuf, ks_out.at[pl.ds(base, _GC)])
        pltpu.sync_copy(qbuf, qd_out.at[pl.ds(base, _GC)])
        pltpu.sync_copy(vbuf, vs_out.at[pl.ds(base, _GC)])
        return 0

    lax.fori_loop(0, _GN, body, 0)


_EDGE_BLK = 1280


def _edge_body(ea_ref, ks_ref, qd_ref, vs_ref,
               wew_ref, web_ref, bew_ref, beb_ref, m_aw_ref, rmat_ref,
               we_ref, s1_ref, wx_ref):
    ea = ea_ref[...]
    Ew = jnp.dot(ea, wew_ref[...], preferred_element_type=f32) + bew_ref[...]
    Eb = jnp.dot(ea, web_ref[...], preferred_element_type=f32) + beb_ref[...]
    g = ks_ref[...] + qd_ref[...]
    edge_w = jnp.maximum(g * Ew + Eb, 0.0)
    we_ref[...] = edge_w
    a = jnp.dot(edge_w, m_aw_ref[...], preferred_element_type=f32)
    w16 = jnp.exp(jnp.clip(a, -CLAMP, CLAMP))
    wexp = jnp.dot(w16, rmat_ref[...], preferred_element_type=f32)
    wx_ref[...] = wexp
    s1_ref[...] = jnp.concatenate([vs_ref[...] * wexp, edge_w * wexp], axis=1)


def _edge(edge_attr, ks, qd, vs, WEw_T, WEb_T, bEw, bEb, M_aw, Rmat):
    n_blk = N_EDGES // _EDGE_BLK
    eb = lambda i: (i, 0)
    z = lambda i: (0, 0)
    return pl.pallas_call(
        _edge_body,
        grid=(n_blk,),
        in_specs=[
            pl.BlockSpec((_EDGE_BLK, IN_DIM), eb),
            pl.BlockSpec((_EDGE_BLK, HD), eb),
            pl.BlockSpec((_EDGE_BLK, HD), eb),
            pl.BlockSpec((_EDGE_BLK, HD), eb),
            pl.BlockSpec((IN_DIM, HD), z),
            pl.BlockSpec((IN_DIM, HD), z),
            pl.BlockSpec((1, HD), z),
            pl.BlockSpec((1, HD), z),
            pl.BlockSpec((HD, 16), z),
            pl.BlockSpec((16, HD), z),
        ],
        out_specs=[
            pl.BlockSpec((_EDGE_BLK, HD), eb),
            pl.BlockSpec((_EDGE_BLK, 2 * HD), eb),
            pl.BlockSpec((_EDGE_BLK, HD), eb),
        ],
        out_shape=[
            jax.ShapeDtypeStruct((N_EDGES, HD), f32),
            jax.ShapeDtypeStruct((N_EDGES, 2 * HD), f32),
            jax.ShapeDtypeStruct((N_EDGES, HD), f32),
        ],
    )(edge_attr, ks, qd, vs, WEw_T, WEb_T, bEw, bEb, M_aw, Rmat)


_EPT = N_EDGES // _NS
_SN = _EPT // _GC
N_PAD = 10240
_RPT = N_PAD // _NS
_NCH = _RPT // _GC


def _zero_acc(acc, dbuf, row0):
    def zrow(rr, _):
        for cc in range(HD // 16):
            dbuf[rr, pl.ds(cc * 16, 16)] = jnp.zeros((16,), f32)
        return 0

    lax.fori_loop(0, _GC, zrow, 0)
    for j in range(_NCH):
        pltpu.sync_copy(dbuf, acc.at[pl.ds(row0 + j * _GC, _GC)])


def _dump_acc(acc, dbuf, out3, c, row0):
    for j in range(_NCH):
        rb = row0 + j * _GC
        pltpu.sync_copy(acc.at[pl.ds(rb, _GC)], dbuf)
        pltpu.sync_copy(dbuf, out3.at[c, pl.ds(rb, _GC)])


@functools.partial(
    pl.kernel,
    out_type=jax.ShapeDtypeStruct((2, N_PAD, HD), f32),
    mesh=_sc_mesh,
    scratch_types=[
        pltpu.VMEM_SHARED((N_PAD, HD), f32),
        pltpu.VMEM((1, _GC), jnp.int32),
        pltpu.VMEM((_GC, HD), f32),
    ],
)
def _sc_scatter_mr(dst_hbm, s1_hbm, out3, acc, idx2, dbuf):
    c = lax.axis_index("c")
    s = lax.axis_index("s")
    row0 = s * _RPT
    col0 = pl.multiple_of(c * HD, HD)

    _zero_acc(acc, dbuf, row0)
    plsc.subcore_barrier()

    def body(i, _):
        base = s * _EPT + i * _GC
        pltpu.sync_copy(dst_hbm.at[pl.ds(base, _GC)], idx2.at[0])
        pltpu.sync_copy(s1_hbm.at[pl.ds(base, _GC), pl.ds(col0, HD)], dbuf)
        pltpu.sync_copy(dbuf, acc.at[idx2.at[0]], add=True)
        return 0

    lax.fori_loop(0, _SN, body, 0)
    plsc.subcore_barrier()
    _dump_acc(acc, dbuf, out3, c, row0)


_EPT2 = N_EDGES // 2 // _NS
_SN2 = _EPT2 // _GC


@functools.partial(
    pl.kernel,
    out_type=jax.ShapeDtypeStruct((2, N_PAD, HD), f32),
    mesh=_sc_mesh,
    scratch_types=[
        pltpu.VMEM_SHARED((N_PAD, HD), f32),
        pltpu.VMEM((1, _GC), jnp.int32),
        pltpu.VMEM((_GC, HD), f32),
    ],
)
def _sc_scatter_wx(dst_hbm, wx_hbm, out3, acc, idx2, dbuf):
    c = lax.axis_index("c")
    s = lax.axis_index("s")
    row0 = s * _RPT

    _zero_acc(acc, dbuf, row0)
    plsc.subcore_barrier()

    def body(i, _):
        base = c * (N_EDGES // 2) + s * _EPT2 + i * _GC
        pltpu.sync_copy(dst_hbm.at[pl.ds(base, _GC)], idx2.at[0])
        pltpu.sync_copy(wx_hbm.at[pl.ds(base, _GC)], dbuf)
        pltpu.sync_copy(dbuf, acc.at[idx2.at[0]], add=True)
        return 0

    lax.fori_loop(0, _SN2, body, 0)
    plsc.subcore_barrier()
    _dump_acc(acc, dbuf, out3, c, row0)


def _final_body(am_ref, ar_ref, aw0_ref, aw1_ref, bve_ref, out_ref):
    inv = 1.0 / (aw0_ref[0] + aw1_ref[0] + 1e-16)
    rv = ar_ref[0] * inv
    out_ref[...] = am_ref[0] * inv + jnp.dot(
        rv, bve_ref[...], preferred_element_type=f32)


_FIN_BLK = 1024


def _final(acc3, accW3, B_ve):
    n_blk = N_PAD // _FIN_BLK
    z = lambda i: (0, 0)
    return pl.pallas_call(
        _final_body,
        grid=(n_blk,),
        in_specs=[
            pl.BlockSpec((1, _FIN_BLK, HD), lambda i: (0, i, 0)),
            pl.BlockSpec((1, _FIN_BLK, HD), lambda i: (1, i, 0)),
            pl.BlockSpec((1, _FIN_BLK, HD), lambda i: (0, i, 0)),
            pl.BlockSpec((1, _FIN_BLK, HD), lambda i: (1, i, 0)),
            pl.BlockSpec((HD, HD), z),
        ],
        out_specs=pl.BlockSpec((_FIN_BLK, HD), lambda i: (i, 0)),
        out_shape=jax.ShapeDtypeStruct((N_PAD, HD), f32),
    )(acc3, acc3, accW3, accW3, B_ve)


def kernel(x, edge_attr, edge_index, WQ, bQ, WK, WE, bE, WV, Aw, VeRow):
    src = edge_index[0].astype(jnp.int32)
    dst = edge_index[1].astype(jnp.int32)

    Wcat = jnp.concatenate([WQ.T, WK.T, WV.T], axis=1)
    bcat = jnp.concatenate([bQ, jnp.zeros((2 * HD,), f32)])[None, :]

    cols = jnp.arange(HD)
    hcol = cols // D
    dcol = cols % D
    perm = hcol * (2 * D) + dcol
    WEt = WE.T
    WEw_T = WEt[:, perm]
    WEb_T = WEt[:, perm + D]
    bEw = bE[perm][None, :]
    bEb = bE[perm + D][None, :]

    M_aw = jnp.zeros((HD, 16), f32).at[cols, hcol].set(Aw[dcol, hcol, 0])
    Rmat = jnp.zeros((16, HD), f32).at[hcol, cols].set(1.0)

    ii = cols[:, None]
    jj = cols[None, :]
    B_ve = jnp.where(ii // D == jj // D,
                     VeRow[ii % D, ii // D, jj % D], 0.0)

    q, k, v = _proj(x, Wcat, bcat)
    ks, qd, vs = _sc_gather(src, dst, q, k, v)
    wE, s1, wx = _edge(edge_attr, ks, qd, vs,
                       WEw_T, WEb_T, bEw, bEb, M_aw, Rmat)
    acc3 = _sc_scatter_mr(dst, s1)
    accW3 = _sc_scatter_wx(dst, wx)
    wV = _final(acc3, accW3, B_ve)
    return wV[:N_NODES].reshape(N_NODES, H, D), wE


# speedup vs baseline: 59.3697x; 1.2971x over previous
import functools

import jax
import jax.numpy as jnp
from jax import lax
from jax.experimental import pallas as pl
from jax.experimental.pallas import tpu as pltpu
from jax.experimental.pallas import tpu_sc as plsc

N_NODES = 10000
N_EDGES = 320000
IN_DIM = 128
H = 8
D = 16
HD = H * D
CLAMP = 5.0

f32 = jnp.float32


_ROWS_BLK = 1000


def _proj_body(x_ref, w_ref, b_ref, q_ref, k_ref, v_ref):
    y = jnp.dot(x_ref[...], w_ref[...], preferred_element_type=f32) + b_ref[...]
    q_ref[...] = y[:, :HD]
    k_ref[...] = y[:, HD:2 * HD]
    v_ref[...] = y[:, 2 * HD:]


def _proj(x, Wcat, bcat):
    n_blk = N_NODES // _ROWS_BLK
    return pl.pallas_call(
        _proj_body,
        grid=(n_blk,),
        in_specs=[
            pl.BlockSpec((_ROWS_BLK, IN_DIM), lambda i: (i, 0)),
            pl.BlockSpec((IN_DIM, 3 * HD), lambda i: (0, 0)),
            pl.BlockSpec((1, 3 * HD), lambda i: (0, 0)),
        ],
        out_specs=[
            pl.BlockSpec((_ROWS_BLK, HD), lambda i: (i, 0)),
            pl.BlockSpec((_ROWS_BLK, HD), lambda i: (i, 0)),
            pl.BlockSpec((_ROWS_BLK, HD), lambda i: (i, 0)),
        ],
        out_shape=[jax.ShapeDtypeStruct((N_NODES, HD), f32)] * 3,
    )(x, Wcat, bcat)


_NC = 2
_NS = 16
_NW = _NC * _NS
_EPW = N_EDGES // _NW
_GC = 80
_GN = _EPW // _GC

_sc_mesh = plsc.VectorSubcoreMesh(core_axis_name="c", subcore_axis_name="s")


@functools.partial(
    pl.kernel,
    out_type=[jax.ShapeDtypeStruct((N_EDGES, HD), f32)] * 3,
    mesh=_sc_mesh,
    scratch_types=[
        pltpu.VMEM((_GC,), jnp.int32),
        pltpu.VMEM((_GC,), jnp.int32),
        pltpu.VMEM((_GC,), jnp.int32),
        pltpu.VMEM((_GC,), jnp.int32),
        pltpu.VMEM((_GC, HD), f32),
        pltpu.VMEM((_GC, HD), f32),
        pltpu.VMEM((_GC, HD), f32),
        pltpu.VMEM((_GC, HD), f32),
        pltpu.VMEM((_GC, HD), f32),
        pltpu.VMEM((_GC, HD), f32),
        pltpu.SemaphoreType.DMA,
        pltpu.SemaphoreType.DMA,
        pltpu.SemaphoreType.DMA,
        pltpu.SemaphoreType.DMA,
    ],
)
def _sc_gather(src_hbm, dst_hbm, q_hbm, k_hbm, v_hbm,
               ks_out, qd_out, vs_out,
               sidxA, didxA, sidxB, didxB,
               kbufA, qbufA, vbufA, kbufB, qbufB, vbufB,
               semA, semB, semWA, semWB):
    w = lax.axis_index("s") * _NC + lax.axis_index("c")
    base0 = w * _EPW

    def body(j, _):
        baseA = base0 + (2 * j) * _GC
        baseB = baseA + _GC
        pltpu.sync_copy(src_hbm.at[pl.ds(baseA, _GC)], sidxA)
        pltpu.sync_copy(dst_hbm.at[pl.ds(baseA, _GC)], didxA)
        ckA = pltpu.async_copy(k_hbm.at[sidxA], kbufA, semA)
        cqA = pltpu.async_copy(q_hbm.at[didxA], qbufA, semA)
        cvA = pltpu.async_copy(v_hbm.at[sidxA], vbufA, semA)
        pltpu.sync_copy(src_hbm.at[pl.ds(baseB, _GC)], sidxB)
        pltpu.sync_copy(dst_hbm.at[pl.ds(baseB, _GC)], didxB)
        ckB = pltpu.async_copy(k_hbm.at[sidxB], kbufB, semB)
        cqB = pltpu.async_copy(q_hbm.at[didxB], qbufB, semB)
        cvB = pltpu.async_copy(v_hbm.at[sidxB], vbufB, semB)
        ckA.wait()
        cqA.wait()
        cvA.wait()
        wkA = pltpu.async_copy(kbufA, ks_out.at[pl.ds(baseA, _GC)], semWA)
        wqA = pltpu.async_copy(qbufA, qd_out.at[pl.ds(baseA, _GC)], semWA)
        wvA = pltpu.async_copy(vbufA, vs_out.at[pl.ds(baseA, _GC)], semWA)
        ckB.wait()
        cqB.wait()
        cvB.wait()
        wkB = pltpu.async_copy(kbufB, ks_out.at[pl.ds(baseB, _GC)], semWB)
        wqB = pltpu.async_copy(qbufB, qd_out.at[pl.ds(baseB, _GC)], semWB)
        wvB = pltpu.async_copy(vbufB, vs_out.at[pl.ds(baseB, _GC)], semWB)
        wkA.wait()
        wqA.wait()
        wvA.wait()
        wkB.wait()
        wqB.wait()
        wvB.wait()
        return 0

    lax.fori_loop(0, _GN // 2, body, 0)

    base = base0 + (_GN - 1) * _GC
    pltpu.sync_copy(src_hbm.at[pl.ds(base, _GC)], sidxA)
    pltpu.sync_copy(dst_hbm.at[pl.ds(base, _GC)], didxA)
    ck = pltpu.async_copy(k_hbm.at[sidxA], kbufA, semA)
    cq = pltpu.async_copy(q_hbm.at[didxA], qbufA, semA)
    cv = pltpu.async_copy(v_hbm.at[sidxA], vbufA, semA)
    ck.wait()
    cq.wait()
    cv.wait()
    pltpu.sync_copy(kbufA, ks_out.at[pl.ds(base, _GC)])
    pltpu.sync_copy(qbufA, qd_out.at[pl.ds(base, _GC)])
    pltpu.sync_copy(vbufA, vs_out.at[pl.ds(base, _GC)])


_EDGE_BLK = 1280


def _edge_body(ea_ref, ks_ref, qd_ref, vs_ref,
               wew_ref, web_ref, bew_ref, beb_ref, m_aw_ref, rmat_ref,
               we_ref, s1_ref, wx_ref):
    ea = ea_ref[...]
    Ew = jnp.dot(ea, wew_ref[...], preferred_element_type=f32) + bew_ref[...]
    Eb = jnp.dot(ea, web_ref[...], preferred_element_type=f32) + beb_ref[...]
    g = ks_ref[...] + qd_ref[...]
    edge_w = jnp.maximum(g * Ew + Eb, 0.0)
    we_ref[...] = edge_w
    a = jnp.dot(edge_w, m_aw_ref[...], preferred_element_type=f32)
    w16 = jnp.exp(jnp.clip(a, -CLAMP, CLAMP))
    wexp = jnp.dot(w16, rmat_ref[...], preferred_element_type=f32)
    wx_ref[...] = wexp
    s1_ref[...] = jnp.concatenate([vs_ref[...] * wexp, edge_w * wexp], axis=1)


def _edge(edge_attr, ks, qd, vs, WEw_T, WEb_T, bEw, bEb, M_aw, Rmat):
    n_blk = N_EDGES // _EDGE_BLK
    eb = lambda i: (i, 0)
    z = lambda i: (0, 0)
    return pl.pallas_call(
        _edge_body,
        grid=(n_blk,),
        in_specs=[
            pl.BlockSpec((_EDGE_BLK, IN_DIM), eb),
            pl.BlockSpec((_EDGE_BLK, HD), eb),
            pl.BlockSpec((_EDGE_BLK, HD), eb),
            pl.BlockSpec((_EDGE_BLK, HD), eb),
            pl.BlockSpec((IN_DIM, HD), z),
            pl.BlockSpec((IN_DIM, HD), z),
            pl.BlockSpec((1, HD), z),
            pl.BlockSpec((1, HD), z),
            pl.BlockSpec((HD, 16), z),
            pl.BlockSpec((16, HD), z),
        ],
        out_specs=[
            pl.BlockSpec((_EDGE_BLK, HD), eb),
            pl.BlockSpec((_EDGE_BLK, 2 * HD), eb),
            pl.BlockSpec((_EDGE_BLK, HD), eb),
        ],
        out_shape=[
            jax.ShapeDtypeStruct((N_EDGES, HD), f32),
            jax.ShapeDtypeStruct((N_EDGES, 2 * HD), f32),
            jax.ShapeDtypeStruct((N_EDGES, HD), f32),
        ],
    )(edge_attr, ks, qd, vs, WEw_T, WEb_T, bEw, bEb, M_aw, Rmat)


_EPT = N_EDGES // _NS
_SN = _EPT // _GC
N_PAD = 10240
_RPT = N_PAD // _NS
_NCH = _RPT // _GC


def _zero_acc(acc, dbuf, row0):
    def zrow(rr, _):
        for cc in range(HD // 16):
            dbuf[rr, pl.ds(cc * 16, 16)] = jnp.zeros((16,), f32)
        return 0

    lax.fori_loop(0, _GC, zrow, 0)
    for j in range(_NCH):
        pltpu.sync_copy(dbuf, acc.at[pl.ds(row0 + j * _GC, _GC)])


def _dump_acc(acc, dbuf, out3, c, row0):
    for j in range(_NCH):
        rb = row0 + j * _GC
        pltpu.sync_copy(acc.at[pl.ds(rb, _GC)], dbuf)
        pltpu.sync_copy(dbuf, out3.at[c, pl.ds(rb, _GC)])


@functools.partial(
    pl.kernel,
    out_type=jax.ShapeDtypeStruct((2, N_PAD, HD), f32),
    mesh=_sc_mesh,
    scratch_types=[
        pltpu.VMEM_SHARED((N_PAD, HD), f32),
        pltpu.VMEM((1, _GC), jnp.int32),
        pltpu.VMEM((1, _GC), jnp.int32),
        pltpu.VMEM((_GC, HD), f32),
        pltpu.VMEM((_GC, HD), f32),
        pltpu.SemaphoreType.DMA,
        pltpu.SemaphoreType.DMA,
        pltpu.SemaphoreType.DMA,
        pltpu.SemaphoreType.DMA,
    ],
)
def _sc_scatter_mr(dst_hbm, s1_hbm, out3, acc, idxA, idxB, dbufA, dbufB,
                   semA, semB, semSA, semSB):
    c = lax.axis_index("c")
    s = lax.axis_index("s")
    row0 = s * _RPT
    col0 = pl.multiple_of(c * HD, HD)

    _zero_acc(acc, dbufA, row0)
    plsc.subcore_barrier()

    def body(j, _):
        baseA = s * _EPT + (2 * j) * _GC
        baseB = baseA + _GC
        pltpu.sync_copy(dst_hbm.at[pl.ds(baseA, _GC)], idxA.at[0])
        cA = pltpu.async_copy(
            s1_hbm.at[pl.ds(baseA, _GC), pl.ds(col0, HD)], dbufA, semA)
        pltpu.sync_copy(dst_hbm.at[pl.ds(baseB, _GC)], idxB.at[0])
        cB = pltpu.async_copy(
            s1_hbm.at[pl.ds(baseB, _GC), pl.ds(col0, HD)], dbufB, semB)
        cA.wait()
        aA = pltpu.async_copy(dbufA, acc.at[idxA.at[0]], semSA, add=True)
        cB.wait()
        aB = pltpu.async_copy(dbufB, acc.at[idxB.at[0]], semSB, add=True)
        aA.wait()
        aB.wait()
        return 0

    lax.fori_loop(0, _SN // 2, body, 0)
    plsc.subcore_barrier()
    _dump_acc(acc, dbufA, out3, c, row0)


_EPT2 = N_EDGES // 2 // _NS
_SN2 = _EPT2 // _GC


@functools.partial(
    pl.kernel,
    out_type=jax.ShapeDtypeStruct((2, N_PAD, HD), f32),
    mesh=_sc_mesh,
    scratch_types=[
        pltpu.VMEM_SHARED((N_PAD, HD), f32),
        pltpu.VMEM((1, _GC), jnp.int32),
        pltpu.VMEM((1, _GC), jnp.int32),
        pltpu.VMEM((_GC, HD), f32),
        pltpu.VMEM((_GC, HD), f32),
        pltpu.SemaphoreType.DMA,
        pltpu.SemaphoreType.DMA,
        pltpu.SemaphoreType.DMA,
        pltpu.SemaphoreType.DMA,
    ],
)
def _sc_scatter_wx(dst_hbm, wx_hbm, out3, acc, idxA, idxB, dbufA, dbufB,
                   semA, semB, semSA, semSB):
    c = lax.axis_index("c")
    s = lax.axis_index("s")
    row0 = s * _RPT
    ebase = c * (N_EDGES // 2) + s * _EPT2

    _zero_acc(acc, dbufA, row0)
    plsc.subcore_barrier()

    def body(j, _):
        baseA = ebase + (2 * j) * _GC
        baseB = baseA + _GC
        pltpu.sync_copy(dst_hbm.at[pl.ds(baseA, _GC)], idxA.at[0])
        cA = pltpu.async_copy(wx_hbm.at[pl.ds(baseA, _GC)], dbufA, semA)
        pltpu.sync_copy(dst_hbm.at[pl.ds(baseB, _GC)], idxB.at[0])
        cB = pltpu.async_copy(wx_hbm.at[pl.ds(baseB, _GC)], dbufB, semB)
        cA.wait()
        aA = pltpu.async_copy(dbufA, acc.at[idxA.at[0]], semSA, add=True)
        cB.wait()
        aB = pltpu.async_copy(dbufB, acc.at[idxB.at[0]], semSB, add=True)
        aA.wait()
        aB.wait()
        return 0

    lax.fori_loop(0, _SN2 // 2, body, 0)

    base = ebase + (_SN2 - 1) * _GC
    pltpu.sync_copy(dst_hbm.at[pl.ds(base, _GC)], idxA.at[0])
    pltpu.sync_copy(wx_hbm.at[pl.ds(base, _GC)], dbufA)
    pltpu.sync_copy(dbufA, acc.at[idxA.at[0]], add=True)

    plsc.subcore_barrier()
    _dump_acc(acc, dbufA, out3, c, row0)


def _final_body(am_ref, ar_ref, aw0_ref, aw1_ref, bve_ref, out_ref):
    inv = 1.0 / (aw0_ref[0] + aw1_ref[0] + 1e-16)
    rv = ar_ref[0] * inv
    out_ref[...] = am_ref[0] * inv + jnp.dot(
        rv, bve_ref[...], preferred_element_type=f32)


_FIN_BLK = 1024


def _final(acc3, accW3, B_ve):
    n_blk = N_PAD // _FIN_BLK
    z = lambda i: (0, 0)
    return pl.pallas_call(
        _final_body,
        grid=(n_blk,),
        in_specs=[
            pl.BlockSpec((1, _FIN_BLK, HD), lambda i: (0, i, 0)),
            pl.BlockSpec((1, _FIN_BLK, HD), lambda i: (1, i, 0)),
            pl.BlockSpec((1, _FIN_BLK, HD), lambda i: (0, i, 0)),
            pl.BlockSpec((1, _FIN_BLK, HD), lambda i: (1, i, 0)),
            pl.BlockSpec((HD, HD), z),
        ],
        out_specs=pl.BlockSpec((_FIN_BLK, HD), lambda i: (i, 0)),
        out_shape=jax.ShapeDtypeStruct((N_PAD, HD), f32),
    )(acc3, acc3, accW3, accW3, B_ve)


def kernel(x, edge_attr, edge_index, WQ, bQ, WK, WE, bE, WV, Aw, VeRow):
    src = edge_index[0].astype(jnp.int32)
    dst = edge_index[1].astype(jnp.int32)

    Wcat = jnp.concatenate([WQ.T, WK.T, WV.T], axis=1)
    bcat = jnp.concatenate([bQ, jnp.zeros((2 * HD,), f32)])[None, :]

    cols = jnp.arange(HD)
    hcol = cols // D
    dcol = cols % D
    perm = hcol * (2 * D) + dcol
    WEt = WE.T
    WEw_T = WEt[:, perm]
    WEb_T = WEt[:, perm + D]
    bEw = bE[perm][None, :]
    bEb = bE[perm + D][None, :]

    M_aw = jnp.zeros((HD, 16), f32).at[cols, hcol].set(Aw[dcol, hcol, 0])
    Rmat = jnp.zeros((16, HD), f32).at[hcol, cols].set(1.0)

    ii = cols[:, None]
    jj = cols[None, :]
    B_ve = jnp.where(ii // D == jj // D,
                     VeRow[ii % D, ii // D, jj % D], 0.0)

    q, k, v = _proj(x, Wcat, bcat)
    ks, qd, vs = _sc_gather(src, dst, q, k, v)
    wE, s1, wx = _edge(edge_attr, ks, qd, vs,
                       WEw_T, WEb_T, bEw, bEb, M_aw, Rmat)
    acc3 = _sc_scatter_mr(dst, s1)
    accW3 = _sc_scatter_wx(dst, wx)
    wV = _final(acc3, accW3, B_ve)
    return wV[:N_NODES].reshape(N_NODES, H, D), wE


# speedup vs baseline: 62.0544x; 1.0452x over previous
import functools

import jax
import jax.numpy as jnp
from jax import lax
from jax.experimental import pallas as pl
from jax.experimental.pallas import tpu as pltpu
from jax.experimental.pallas import tpu_sc as plsc

N_NODES = 10000
N_EDGES = 320000
IN_DIM = 128
H = 8
D = 16
HD = H * D
CLAMP = 5.0

f32 = jnp.float32


_ROWS_BLK = 1000


def _proj_body(x_ref, w_ref, b_ref, q_ref, k_ref, v_ref):
    y = jnp.dot(x_ref[...], w_ref[...], preferred_element_type=f32) + b_ref[...]
    q_ref[...] = y[:, :HD]
    k_ref[...] = y[:, HD:2 * HD]
    v_ref[...] = y[:, 2 * HD:]


def _proj(x, Wcat, bcat):
    n_blk = N_NODES // _ROWS_BLK
    return pl.pallas_call(
        _proj_body,
        grid=(n_blk,),
        in_specs=[
            pl.BlockSpec((_ROWS_BLK, IN_DIM), lambda i: (i, 0)),
            pl.BlockSpec((IN_DIM, 3 * HD), lambda i: (0, 0)),
            pl.BlockSpec((1, 3 * HD), lambda i: (0, 0)),
        ],
        out_specs=[
            pl.BlockSpec((_ROWS_BLK, HD), lambda i: (i, 0)),
            pl.BlockSpec((_ROWS_BLK, HD), lambda i: (i, 0)),
            pl.BlockSpec((_ROWS_BLK, HD), lambda i: (i, 0)),
        ],
        out_shape=[jax.ShapeDtypeStruct((N_NODES, HD), f32)] * 3,
    )(x, Wcat, bcat)


_NC = 2
_NS = 16
_NW = _NC * _NS
_EPW = N_EDGES // _NW
_GC = 80
_GN = _EPW // _GC

_sc_mesh = plsc.VectorSubcoreMesh(core_axis_name="c", subcore_axis_name="s")


def _add_rows(buf_k, buf_q):
    def arow(rr, _):
        for cc in range(HD // 16):
            sl = pl.ds(cc * 16, 16)
            buf_k[rr, sl] = buf_k[rr, sl] + buf_q[rr, sl]
        return 0

    lax.fori_loop(0, _GC, arow, 0)


@functools.partial(
    pl.kernel,
    out_type=[jax.ShapeDtypeStruct((N_EDGES, HD), f32)] * 2,
    mesh=_sc_mesh,
    scratch_types=[
        pltpu.VMEM((_GC,), jnp.int32),
        pltpu.VMEM((_GC,), jnp.int32),
        pltpu.VMEM((_GC,), jnp.int32),
        pltpu.VMEM((_GC,), jnp.int32),
        pltpu.VMEM((_GC, HD), f32),
        pltpu.VMEM((_GC, HD), f32),
        pltpu.VMEM((_GC, HD), f32),
        pltpu.VMEM((_GC, HD), f32),
        pltpu.VMEM((_GC, HD), f32),
        pltpu.VMEM((_GC, HD), f32),
        pltpu.SemaphoreType.DMA,
        pltpu.SemaphoreType.DMA,
        pltpu.SemaphoreType.DMA,
        pltpu.SemaphoreType.DMA,
    ],
)
def _sc_gather(src_hbm, dst_hbm, q_hbm, k_hbm, v_hbm,
               g_out, vs_out,
               sidxA, didxA, sidxB, didxB,
               kbufA, qbufA, vbufA, kbufB, qbufB, vbufB,
               semA, semB, semWA, semWB):
    w = lax.axis_index("s") * _NC + lax.axis_index("c")
    base0 = w * _EPW

    def body(j, _):
        baseA = base0 + (2 * j) * _GC
        baseB = baseA + _GC
        pltpu.sync_copy(src_hbm.at[pl.ds(baseA, _GC)], sidxA)
        pltpu.sync_copy(dst_hbm.at[pl.ds(baseA, _GC)], didxA)
        ckA = pltpu.async_copy(k_hbm.at[sidxA], kbufA, semA)
        cqA = pltpu.async_copy(q_hbm.at[didxA], qbufA, semA)
        cvA = pltpu.async_copy(v_hbm.at[sidxA], vbufA, semA)
        pltpu.sync_copy(src_hbm.at[pl.ds(baseB, _GC)], sidxB)
        pltpu.sync_copy(dst_hbm.at[pl.ds(baseB, _GC)], didxB)
        ckB = pltpu.async_copy(k_hbm.at[sidxB], kbufB, semB)
        cqB = pltpu.async_copy(q_hbm.at[didxB], qbufB, semB)
        cvB = pltpu.async_copy(v_hbm.at[sidxB], vbufB, semB)
        ckA.wait()
        cqA.wait()
        _add_rows(kbufA, qbufA)
        cvA.wait()
        wkA = pltpu.async_copy(kbufA, g_out.at[pl.ds(baseA, _GC)], semWA)
        wvA = pltpu.async_copy(vbufA, vs_out.at[pl.ds(baseA, _GC)], semWA)
        ckB.wait()
        cqB.wait()
        _add_rows(kbufB, qbufB)
        cvB.wait()
        wkB = pltpu.async_copy(kbufB, g_out.at[pl.ds(baseB, _GC)], semWB)
        wvB = pltpu.async_copy(vbufB, vs_out.at[pl.ds(baseB, _GC)], semWB)
        wkA.wait()
        wvA.wait()
        wkB.wait()
        wvB.wait()
        return 0

    lax.fori_loop(0, _GN // 2, body, 0)

    base = base0 + (_GN - 1) * _GC
    pltpu.sync_copy(src_hbm.at[pl.ds(base, _GC)], sidxA)
    pltpu.sync_copy(dst_hbm.at[pl.ds(base, _GC)], didxA)
    ck = pltpu.async_copy(k_hbm.at[sidxA], kbufA, semA)
    cq = pltpu.async_copy(q_hbm.at[didxA], qbufA, semA)
    cv = pltpu.async_copy(v_hbm.at[sidxA], vbufA, semA)
    ck.wait()
    cq.wait()
    _add_rows(kbufA, qbufA)
    cv.wait()
    pltpu.sync_copy(kbufA, g_out.at[pl.ds(base, _GC)])
    pltpu.sync_copy(vbufA, vs_out.at[pl.ds(base, _GC)])


_EDGE_BLK = 1280


def _edge_body(ea_ref, g_ref, vs_ref,
               wew_ref, web_ref, bew_ref, beb_ref, m_aw_ref, rmat_ref,
               we_ref, s1_ref, wx_ref):
    ea = ea_ref[...]
    Ew = jnp.dot(ea, wew_ref[...], preferred_element_type=f32) + bew_ref[...]
    Eb = jnp.dot(ea, web_ref[...], preferred_element_type=f32) + beb_ref[...]
    edge_w = jnp.maximum(g_ref[...] * Ew + Eb, 0.0)
    we_ref[...] = edge_w
    a = jnp.dot(edge_w, m_aw_ref[...], preferred_element_type=f32)
    w16 = jnp.exp(jnp.clip(a, -CLAMP, CLAMP))
    wexp = jnp.dot(w16, rmat_ref[...], preferred_element_type=f32)
    wx_ref[...] = wexp
    s1_ref[...] = jnp.concatenate([vs_ref[...] * wexp, edge_w * wexp], axis=1)


def _edge(edge_attr, g, vs, WEw_T, WEb_T, bEw, bEb, M_aw, Rmat):
    n_blk = N_EDGES // _EDGE_BLK
    eb = lambda i: (i, 0)
    z = lambda i: (0, 0)
    return pl.pallas_call(
        _edge_body,
        grid=(n_blk,),
        in_specs=[
            pl.BlockSpec((_EDGE_BLK, IN_DIM), eb),
            pl.BlockSpec((_EDGE_BLK, HD), eb),
            pl.BlockSpec((_EDGE_BLK, HD), eb),
            pl.BlockSpec((IN_DIM, HD), z),
            pl.BlockSpec((IN_DIM, HD), z),
            pl.BlockSpec((1, HD), z),
            pl.BlockSpec((1, HD), z),
            pl.BlockSpec((HD, 16), z),
            pl.BlockSpec((16, HD), z),
        ],
        out_specs=[
            pl.BlockSpec((_EDGE_BLK, HD), eb),
            pl.BlockSpec((_EDGE_BLK, 2 * HD), eb),
            pl.BlockSpec((_EDGE_BLK, HD), eb),
        ],
        out_shape=[
            jax.ShapeDtypeStruct((N_EDGES, HD), f32),
            jax.ShapeDtypeStruct((N_EDGES, 2 * HD), f32),
            jax.ShapeDtypeStruct((N_EDGES, HD), f32),
        ],
    )(edge_attr, g, vs, WEw_T, WEb_T, bEw, bEb, M_aw, Rmat)


_EPT = N_EDGES // _NS
_SN = _EPT // _GC
N_PAD = 10240
_RPT = N_PAD // _NS
_NCH = _RPT // _GC


def _zero_acc(acc, dbuf, row0):
    def zrow(rr, _):
        for cc in range(HD // 16):
            dbuf[rr, pl.ds(cc * 16, 16)] = jnp.zeros((16,), f32)
        return 0

    lax.fori_loop(0, _GC, zrow, 0)
    for j in range(_NCH):
        pltpu.sync_copy(dbuf, acc.at[pl.ds(row0 + j * _GC, _GC)])


def _dump_acc(acc, dbuf, out3, c, row0):
    for j in range(_NCH):
        rb = row0 + j * _GC
        pltpu.sync_copy(acc.at[pl.ds(rb, _GC)], dbuf)
        pltpu.sync_copy(dbuf, out3.at[c, pl.ds(rb, _GC)])


@functools.partial(
    pl.kernel,
    out_type=jax.ShapeDtypeStruct((2, N_PAD, HD), f32),
    mesh=_sc_mesh,
    scratch_types=[
        pltpu.VMEM_SHARED((N_PAD, HD), f32),
        pltpu.VMEM((1, _GC), jnp.int32),
        pltpu.VMEM((1, _GC), jnp.int32),
        pltpu.VMEM((_GC, HD), f32),
        pltpu.VMEM((_GC, HD), f32),
        pltpu.SemaphoreType.DMA,
        pltpu.SemaphoreType.DMA,
        pltpu.SemaphoreType.DMA,
        pltpu.SemaphoreType.DMA,
    ],
)
def _sc_scatter_mr(dst_hbm, s1_hbm, out3, acc, idxA, idxB, dbufA, dbufB,
                   semA, semB, semSA, semSB):
    c = lax.axis_index("c")
    s = lax.axis_index("s")
    row0 = s * _RPT
    col0 = pl.multiple_of(c * HD, HD)

    _zero_acc(acc, dbufA, row0)
    plsc.subcore_barrier()

    def body(j, _):
        baseA = s * _EPT + (2 * j) * _GC
        baseB = baseA + _GC
        pltpu.sync_copy(dst_hbm.at[pl.ds(baseA, _GC)], idxA.at[0])
        cA = pltpu.async_copy(
            s1_hbm.at[pl.ds(baseA, _GC), pl.ds(col0, HD)], dbufA, semA)
        pltpu.sync_copy(dst_hbm.at[pl.ds(baseB, _GC)], idxB.at[0])
        cB = pltpu.async_copy(
            s1_hbm.at[pl.ds(baseB, _GC), pl.ds(col0, HD)], dbufB, semB)
        cA.wait()
        aA = pltpu.async_copy(dbufA, acc.at[idxA.at[0]], semSA, add=True)
        cB.wait()
        aB = pltpu.async_copy(dbufB, acc.at[idxB.at[0]], semSB, add=True)
        aA.wait()
        aB.wait()
        return 0

    lax.fori_loop(0, _SN // 2, body, 0)
    plsc.subcore_barrier()
    _dump_acc(acc, dbufA, out3, c, row0)


_EPT2 = N_EDGES // 2 // _NS
_SN2 = _EPT2 // _GC


@functools.partial(
    pl.kernel,
    out_type=jax.ShapeDtypeStruct((2, N_PAD, HD), f32),
    mesh=_sc_mesh,
    scratch_types=[
        pltpu.VMEM_SHARED((N_PAD, HD), f32),
        pltpu.VMEM((1, _GC), jnp.int32),
        pltpu.VMEM((1, _GC), jnp.int32),
        pltpu.VMEM((_GC, HD), f32),
        pltpu.VMEM((_GC, HD), f32),
        pltpu.SemaphoreType.DMA,
        pltpu.SemaphoreType.DMA,
        pltpu.SemaphoreType.DMA,
        pltpu.SemaphoreType.DMA,
    ],
)
def _sc_scatter_wx(dst_hbm, wx_hbm, out3, acc, idxA, idxB, dbufA, dbufB,
                   semA, semB, semSA, semSB):
    c = lax.axis_index("c")
    s = lax.axis_index("s")
    row0 = s * _RPT
    ebase = c * (N_EDGES // 2) + s * _EPT2

    _zero_acc(acc, dbufA, row0)
    plsc.subcore_barrier()

    def body(j, _):
        baseA = ebase + (2 * j) * _GC
        baseB = baseA + _GC
        pltpu.sync_copy(dst_hbm.at[pl.ds(baseA, _GC)], idxA.at[0])
        cA = pltpu.async_copy(wx_hbm.at[pl.ds(baseA, _GC)], dbufA, semA)
        pltpu.sync_copy(dst_hbm.at[pl.ds(baseB, _GC)], idxB.at[0])
        cB = pltpu.async_copy(wx_hbm.at[pl.ds(baseB, _GC)], dbufB, semB)
        cA.wait()
        aA = pltpu.async_copy(dbufA, acc.at[idxA.at[0]], semSA, add=True)
        cB.wait()
        aB = pltpu.async_copy(dbufB, acc.at[idxB.at[0]], semSB, add=True)
        aA.wait()
        aB.wait()
        return 0

    lax.fori_loop(0, _SN2 // 2, body, 0)

    base = ebase + (_SN2 - 1) * _GC
    pltpu.sync_copy(dst_hbm.at[pl.ds(base, _GC)], idxA.at[0])
    pltpu.sync_copy(wx_hbm.at[pl.ds(base, _GC)], dbufA)
    pltpu.sync_copy(dbufA, acc.at[idxA.at[0]], add=True)

    plsc.subcore_barrier()
    _dump_acc(acc, dbufA, out3, c, row0)


def _final_body(am_ref, ar_ref, aw0_ref, aw1_ref, bve_ref, out_ref):
    inv = 1.0 / (aw0_ref[0] + aw1_ref[0] + 1e-16)
    rv = ar_ref[0] * inv
    out_ref[...] = am_ref[0] * inv + jnp.dot(
        rv, bve_ref[...], preferred_element_type=f32)


_FIN_BLK = 1024


def _final(acc3, accW3, B_ve):
    n_blk = N_PAD // _FIN_BLK
    z = lambda i: (0, 0)
    return pl.pallas_call(
        _final_body,
        grid=(n_blk,),
        in_specs=[
            pl.BlockSpec((1, _FIN_BLK, HD), lambda i: (0, i, 0)),
            pl.BlockSpec((1, _FIN_BLK, HD), lambda i: (1, i, 0)),
            pl.BlockSpec((1, _FIN_BLK, HD), lambda i: (0, i, 0)),
            pl.BlockSpec((1, _FIN_BLK, HD), lambda i: (1, i, 0)),
            pl.BlockSpec((HD, HD), z),
        ],
        out_specs=pl.BlockSpec((_FIN_BLK, HD), lambda i: (i, 0)),
        out_shape=jax.ShapeDtypeStruct((N_PAD, HD), f32),
    )(acc3, acc3, accW3, accW3, B_ve)


def kernel(x, edge_attr, edge_index, WQ, bQ, WK, WE, bE, WV, Aw, VeRow):
    src = edge_index[0].astype(jnp.int32)
    dst = edge_index[1].astype(jnp.int32)

    Wcat = jnp.concatenate([WQ.T, WK.T, WV.T], axis=1)
    bcat = jnp.concatenate([bQ, jnp.zeros((2 * HD,), f32)])[None, :]

    cols = jnp.arange(HD)
    hcol = cols // D
    dcol = cols % D
    perm = hcol * (2 * D) + dcol
    WEt = WE.T
    WEw_T = WEt[:, perm]
    WEb_T = WEt[:, perm + D]
    bEw = bE[perm][None, :]
    bEb = bE[perm + D][None, :]

    M_aw = jnp.zeros((HD, 16), f32).at[cols, hcol].set(Aw[dcol, hcol, 0])
    Rmat = jnp.zeros((16, HD), f32).at[hcol, cols].set(1.0)

    ii = cols[:, None]
    jj = cols[None, :]
    B_ve = jnp.where(ii // D == jj // D,
                     VeRow[ii % D, ii // D, jj % D], 0.0)

    q, k, v = _proj(x, Wcat, bcat)
    g, vs = _sc_gather(src, dst, q, k, v)
    wE, s1, wx = _edge(edge_attr, g, vs,
                       WEw_T, WEb_T, bEw, bEb, M_aw, Rmat)
    acc3 = _sc_scatter_mr(dst, s1)
    accW3 = _sc_scatter_wx(dst, wx)
    wV = _final(acc3, accW3, B_ve)
    return wV[:N_NODES].reshape(N_NODES, H, D), wE


# speedup vs baseline: 64.7446x; 1.0434x over previous
import functools

import jax
import jax.numpy as jnp
from jax import lax
from jax.experimental import pallas as pl
from jax.experimental.pallas import tpu as pltpu
from jax.experimental.pallas import tpu_sc as plsc

N_NODES = 10000
N_EDGES = 320000
IN_DIM = 128
H = 8
D = 16
HD = H * D
CLAMP = 5.0

f32 = jnp.float32


_ROWS_BLK = 1000


bf16 = jnp.bfloat16


def _proj_body(x_ref, w_ref, b_ref, q_ref, kv_ref):
    y = jnp.dot(x_ref[...], w_ref[...], preferred_element_type=f32) + b_ref[...]
    q_ref[...] = y[:, :HD]
    kb = jax.lax.bitcast_convert_type(
        y[:, HD:2 * HD].astype(bf16), jnp.uint16).astype(jnp.uint32)
    vb = jax.lax.bitcast_convert_type(
        y[:, 2 * HD:].astype(bf16), jnp.uint16).astype(jnp.uint32)
    kv_ref[...] = jax.lax.bitcast_convert_type(kb | (vb << 16), jnp.int32)


def _proj(x, Wcat, bcat):
    n_blk = N_NODES // _ROWS_BLK
    return pl.pallas_call(
        _proj_body,
        grid=(n_blk,),
        in_specs=[
            pl.BlockSpec((_ROWS_BLK, IN_DIM), lambda i: (i, 0)),
            pl.BlockSpec((IN_DIM, 3 * HD), lambda i: (0, 0)),
            pl.BlockSpec((1, 3 * HD), lambda i: (0, 0)),
        ],
        out_specs=[
            pl.BlockSpec((_ROWS_BLK, HD), lambda i: (i, 0)),
            pl.BlockSpec((_ROWS_BLK, HD), lambda i: (i, 0)),
        ],
        out_shape=[
            jax.ShapeDtypeStruct((N_NODES, HD), f32),
            jax.ShapeDtypeStruct((N_NODES, HD), jnp.int32),
        ],
    )(x, Wcat, bcat)


_NC = 2
_NS = 16
_NW = _NC * _NS
_EPW = N_EDGES // _NW
_GC = 80
_GN = _EPW // _GC

_sc_mesh = plsc.VectorSubcoreMesh(core_axis_name="c", subcore_axis_name="s")


@functools.partial(
    pl.kernel,
    out_type=[
        jax.ShapeDtypeStruct((N_EDGES, HD), f32),
        jax.ShapeDtypeStruct((N_EDGES, HD), jnp.int32),
    ],
    mesh=_sc_mesh,
    scratch_types=[
        pltpu.VMEM((_GC,), jnp.int32),
        pltpu.VMEM((_GC,), jnp.int32),
        pltpu.VMEM((_GC,), jnp.int32),
        pltpu.VMEM((_GC,), jnp.int32),
        pltpu.VMEM((_GC, HD), f32),
        pltpu.VMEM((_GC, HD), jnp.int32),
        pltpu.VMEM((_GC, HD), f32),
        pltpu.VMEM((_GC, HD), jnp.int32),
        pltpu.SemaphoreType.DMA,
        pltpu.SemaphoreType.DMA,
        pltpu.SemaphoreType.DMA,
        pltpu.SemaphoreType.DMA,
    ],
)
def _sc_gather(src_hbm, dst_hbm, q_hbm, kv_hbm,
               qd_out, kv_out,
               sidxA, didxA, sidxB, didxB,
               qbufA, kvbufA, qbufB, kvbufB,
               semA, semB, semWA, semWB):
    w = lax.axis_index("s") * _NC + lax.axis_index("c")
    base0 = w * _EPW

    def body(j, _):
        baseA = base0 + (2 * j) * _GC
        baseB = baseA + _GC
        pltpu.sync_copy(src_hbm.at[pl.ds(baseA, _GC)], sidxA)
        pltpu.sync_copy(dst_hbm.at[pl.ds(baseA, _GC)], didxA)
        ckA = pltpu.async_copy(kv_hbm.at[sidxA], kvbufA, semA)
        cqA = pltpu.async_copy(q_hbm.at[didxA], qbufA, semA)
        pltpu.sync_copy(src_hbm.at[pl.ds(baseB, _GC)], sidxB)
        pltpu.sync_copy(dst_hbm.at[pl.ds(baseB, _GC)], didxB)
        ckB = pltpu.async_copy(kv_hbm.at[sidxB], kvbufB, semB)
        cqB = pltpu.async_copy(q_hbm.at[didxB], qbufB, semB)
        ckA.wait()
        cqA.wait()
        wkA = pltpu.async_copy(kvbufA, kv_out.at[pl.ds(baseA, _GC)], semWA)
        wqA = pltpu.async_copy(qbufA, qd_out.at[pl.ds(baseA, _GC)], semWA)
        ckB.wait()
        cqB.wait()
        wkB = pltpu.async_copy(kvbufB, kv_out.at[pl.ds(baseB, _GC)], semWB)
        wqB = pltpu.async_copy(qbufB, qd_out.at[pl.ds(baseB, _GC)], semWB)
        wkA.wait()
        wqA.wait()
        wkB.wait()
        wqB.wait()
        return 0

    lax.fori_loop(0, _GN // 2, body, 0)

    base = base0 + (_GN - 1) * _GC
    pltpu.sync_copy(src_hbm.at[pl.ds(base, _GC)], sidxA)
    pltpu.sync_copy(dst_hbm.at[pl.ds(base, _GC)], didxA)
    ck = pltpu.async_copy(kv_hbm.at[sidxA], kvbufA, semA)
    cq = pltpu.async_copy(q_hbm.at[didxA], qbufA, semA)
    ck.wait()
    cq.wait()
    pltpu.sync_copy(kvbufA, kv_out.at[pl.ds(base, _GC)])
    pltpu.sync_copy(qbufA, qd_out.at[pl.ds(base, _GC)])


_EDGE_BLK = 1280


def _edge_body(ea_ref, qd_ref, kv_ref,
               wew_ref, web_ref, bew_ref, beb_ref, m_aw_ref, rmat_ref,
               we_ref, s1_ref, wx_ref):
    ea = ea_ref[...]
    Ew = jnp.dot(ea, wew_ref[...], preferred_element_type=f32) + bew_ref[...]
    Eb = jnp.dot(ea, web_ref[...], preferred_element_type=f32) + beb_ref[...]
    kv = kv_ref[...]
    ks = jax.lax.bitcast_convert_type(jax.lax.shift_left(kv, 16), f32)
    vs = jax.lax.bitcast_convert_type(
        jnp.bitwise_and(kv, jnp.int32(-65536)), f32)
    g = ks + qd_ref[...]
    edge_w = jnp.maximum(g * Ew + Eb, 0.0)
    we_ref[...] = edge_w
    a = jnp.dot(edge_w, m_aw_ref[...], preferred_element_type=f32)
    w16 = jnp.exp(jnp.clip(a, -CLAMP, CLAMP))
    wexp = jnp.dot(w16, rmat_ref[...], preferred_element_type=f32)
    wx_ref[...] = wexp
    s1_ref[...] = jnp.concatenate([vs * wexp, edge_w * wexp], axis=1)


def _edge(edge_attr, qd, kv, WEw_T, WEb_T, bEw, bEb, M_aw, Rmat):
    n_blk = N_EDGES // _EDGE_BLK
    eb = lambda i: (i, 0)
    z = lambda i: (0, 0)
    return pl.pallas_call(
        _edge_body,
        grid=(n_blk,),
        in_specs=[
            pl.BlockSpec((_EDGE_BLK, IN_DIM), eb),
            pl.BlockSpec((_EDGE_BLK, HD), eb),
            pl.BlockSpec((_EDGE_BLK, HD), eb),
            pl.BlockSpec((IN_DIM, HD), z),
            pl.BlockSpec((IN_DIM, HD), z),
            pl.BlockSpec((1, HD), z),
            pl.BlockSpec((1, HD), z),
            pl.BlockSpec((HD, 16), z),
            pl.BlockSpec((16, HD), z),
        ],
        out_specs=[
            pl.BlockSpec((_EDGE_BLK, HD), eb),
            pl.BlockSpec((_EDGE_BLK, 2 * HD), eb),
            pl.BlockSpec((_EDGE_BLK, HD), eb),
        ],
        out_shape=[
            jax.ShapeDtypeStruct((N_EDGES, HD), f32),
            jax.ShapeDtypeStruct((N_EDGES, 2 * HD), f32),
            jax.ShapeDtypeStruct((N_EDGES, HD), f32),
        ],
    )(edge_attr, qd, kv, WEw_T, WEb_T, bEw, bEb, M_aw, Rmat)


_EPT = N_EDGES // _NS
_SN = _EPT // _GC
N_PAD = 10240
_RPT = N_PAD // _NS
_NCH = _RPT // _GC


def _zero_acc(acc, dbuf, row0):
    def zrow(rr, _):
        for cc in range(HD // 16):
            dbuf[rr, pl.ds(cc * 16, 16)] = jnp.zeros((16,), f32)
        return 0

    lax.fori_loop(0, _GC, zrow, 0)
    for j in range(_NCH):
        pltpu.sync_copy(dbuf, acc.at[pl.ds(row0 + j * _GC, _GC)])


def _dump_acc(acc, dbuf, out3, c, row0):
    for j in range(_NCH):
        rb = row0 + j * _GC
        pltpu.sync_copy(acc.at[pl.ds(rb, _GC)], dbuf)
        pltpu.sync_copy(dbuf, out3.at[c, pl.ds(rb, _GC)])


@functools.partial(
    pl.kernel,
    out_type=jax.ShapeDtypeStruct((2, N_PAD, HD), f32),
    mesh=_sc_mesh,
    scratch_types=[
        pltpu.VMEM_SHARED((N_PAD, HD), f32),
        pltpu.VMEM((1, _GC), jnp.int32),
        pltpu.VMEM((1, _GC), jnp.int32),
        pltpu.VMEM((_GC, HD), f32),
        pltpu.VMEM((_GC, HD), f32),
        pltpu.SemaphoreType.DMA,
        pltpu.SemaphoreType.DMA,
        pltpu.SemaphoreType.DMA,
        pltpu.SemaphoreType.DMA,
    ],
)
def _sc_scatter_mr(dst_hbm, s1_hbm, out3, acc, idxA, idxB, dbufA, dbufB,
                   semA, semB, semSA, semSB):
    c = lax.axis_index("c")
    s = lax.axis_index("s")
    row0 = s * _RPT
    col0 = pl.multiple_of(c * HD, HD)

    _zero_acc(acc, dbufA, row0)
    plsc.subcore_barrier()

    def body(j, _):
        baseA = s * _EPT + (2 * j) * _GC
        baseB = baseA + _GC
        pltpu.sync_copy(dst_hbm.at[pl.ds(baseA, _GC)], idxA.at[0])
        cA = pltpu.async_copy(
            s1_hbm.at[pl.ds(baseA, _GC), pl.ds(col0, HD)], dbufA, semA)
        pltpu.sync_copy(dst_hbm.at[pl.ds(baseB, _GC)], idxB.at[0])
        cB = pltpu.async_copy(
            s1_hbm.at[pl.ds(baseB, _GC), pl.ds(col0, HD)], dbufB, semB)
        cA.wait()
        aA = pltpu.async_copy(dbufA, acc.at[idxA.at[0]], semSA, add=True)
        cB.wait()
        aB = pltpu.async_copy(dbufB, acc.at[idxB.at[0]], semSB, add=True)
        aA.wait()
        aB.wait()
        return 0

    lax.fori_loop(0, _SN // 2, body, 0)
    plsc.subcore_barrier()
    _dump_acc(acc, dbufA, out3, c, row0)


_EPT2 = N_EDGES // 2 // _NS
_SN2 = _EPT2 // _GC


@functools.partial(
    pl.kernel,
    out_type=jax.ShapeDtypeStruct((2, N_PAD, HD), f32),
    mesh=_sc_mesh,
    scratch_types=[
        pltpu.VMEM_SHARED((N_PAD, HD), f32),
        pltpu.VMEM((1, _GC), jnp.int32),
        pltpu.VMEM((1, _GC), jnp.int32),
        pltpu.VMEM((_GC, HD), f32),
        pltpu.VMEM((_GC, HD), f32),
        pltpu.SemaphoreType.DMA,
        pltpu.SemaphoreType.DMA,
        pltpu.SemaphoreType.DMA,
        pltpu.SemaphoreType.DMA,
    ],
)
def _sc_scatter_wx(dst_hbm, wx_hbm, out3, acc, idxA, idxB, dbufA, dbufB,
                   semA, semB, semSA, semSB):
    c = lax.axis_index("c")
    s = lax.axis_index("s")
    row0 = s * _RPT
    ebase = c * (N_EDGES // 2) + s * _EPT2

    _zero_acc(acc, dbufA, row0)
    plsc.subcore_barrier()

    def body(j, _):
        baseA = ebase + (2 * j) * _GC
        baseB = baseA + _GC
        pltpu.sync_copy(dst_hbm.at[pl.ds(baseA, _GC)], idxA.at[0])
        cA = pltpu.async_copy(wx_hbm.at[pl.ds(baseA, _GC)], dbufA, semA)
        pltpu.sync_copy(dst_hbm.at[pl.ds(baseB, _GC)], idxB.at[0])
        cB = pltpu.async_copy(wx_hbm.at[pl.ds(baseB, _GC)], dbufB, semB)
        cA.wait()
        aA = pltpu.async_copy(dbufA, acc.at[idxA.at[0]], semSA, add=True)
        cB.wait()
        aB = pltpu.async_copy(dbufB, acc.at[idxB.at[0]], semSB, add=True)
        aA.wait()
        aB.wait()
        return 0

    lax.fori_loop(0, _SN2 // 2, body, 0)

    base = ebase + (_SN2 - 1) * _GC
    pltpu.sync_copy(dst_hbm.at[pl.ds(base, _GC)], idxA.at[0])
    pltpu.sync_copy(wx_hbm.at[pl.ds(base, _GC)], dbufA)
    pltpu.sync_copy(dbufA, acc.at[idxA.at[0]], add=True)

    plsc.subcore_barrier()
    _dump_acc(acc, dbufA, out3, c, row0)


def _final_body(am_ref, ar_ref, aw0_ref, aw1_ref, bve_ref, out_ref):
    inv = 1.0 / (aw0_ref[0] + aw1_ref[0] + 1e-16)
    rv = ar_ref[0] * inv
    out_ref[...] = am_ref[0] * inv + jnp.dot(
        rv, bve_ref[...], preferred_element_type=f32)


_FIN_BLK = 1024


def _final(acc3, accW3, B_ve):
    n_blk = N_PAD // _FIN_BLK
    z = lambda i: (0, 0)
    return pl.pallas_call(
        _final_body,
        grid=(n_blk,),
        in_specs=[
            pl.BlockSpec((1, _FIN_BLK, HD), lambda i: (0, i, 0)),
            pl.BlockSpec((1, _FIN_BLK, HD), lambda i: (1, i, 0)),
            pl.BlockSpec((1, _FIN_BLK, HD), lambda i: (0, i, 0)),
            pl.BlockSpec((1, _FIN_BLK, HD), lambda i: (1, i, 0)),
            pl.BlockSpec((HD, HD), z),
        ],
        out_specs=pl.BlockSpec((_FIN_BLK, HD), lambda i: (i, 0)),
        out_shape=jax.ShapeDtypeStruct((N_PAD, HD), f32),
    )(acc3, acc3, accW3, accW3, B_ve)


def kernel(x, edge_attr, edge_index, WQ, bQ, WK, WE, bE, WV, Aw, VeRow):
    src = edge_index[0].astype(jnp.int32)
    dst = edge_index[1].astype(jnp.int32)

    Wcat = jnp.concatenate([WQ.T, WK.T, WV.T], axis=1)
    bcat = jnp.concatenate([bQ, jnp.zeros((2 * HD,), f32)])[None, :]

    cols = jnp.arange(HD)
    hcol = cols // D
    dcol = cols % D
    perm = hcol * (2 * D) + dcol
    WEt = WE.T
    WEw_T = WEt[:, perm]
    WEb_T = WEt[:, perm + D]
    bEw = bE[perm][None, :]
    bEb = bE[perm + D][None, :]

    M_aw = jnp.zeros((HD, 16), f32).at[cols, hcol].set(Aw[dcol, hcol, 0])
    Rmat = jnp.zeros((16, HD), f32).at[hcol, cols].set(1.0)

    ii = cols[:, None]
    jj = cols[None, :]
    B_ve = jnp.where(ii // D == jj // D,
                     VeRow[ii % D, ii // D, jj % D], 0.0)

    q, kv = _proj(x, Wcat, bcat)
    qd, kve = _sc_gather(src, dst, q, kv)
    wE, s1, wx = _edge(edge_attr, qd, kve,
                       WEw_T, WEb_T, bEw, bEb, M_aw, Rmat)
    acc3 = _sc_scatter_mr(dst, s1)
    accW3 = _sc_scatter_wx(dst, wx)
    wV = _final(acc3, accW3, B_ve)
    return wV[:N_NODES].reshape(N_NODES, H, D), wE
